# baseline (speedup 1.0000x reference)
import functools
import jax
import jax.numpy as jnp
from jax import lax
from jax.experimental import pallas as pl
from jax.experimental.pallas import tpu as pltpu
from jax.experimental.pallas import tpu_sc as plsc

N = 10000
E = 320000
C = 128
NW = 32
NPW = 320
NPAD = NW * NPW
CH = 2000
NGR = CH // 16
GB = 128
ROWB = 1024
NBLK = NPAD // ROWB


def _leaky(x):
    return jnp.where(x >= 0, x, 0.01 * x)


def _tc1_body(h_ref, pos_ref, A1_ref, b1_ref, A2_ref, b2_ref, G3_ref, AH_ref,
              bf_ref, u_ref, v_ref):
    x = h_ref[...]
    p8 = pos_ref[...]
    t1 = _leaky(jnp.dot(x, A1_ref[...], preferred_element_type=jnp.float32)
                + b1_ref[...])
    d8 = jnp.tanh(jnp.dot(t1, A2_ref[...], preferred_element_type=jnp.float32)
                  + b2_ref[...])
    u_ref[...] = (jnp.dot(p8, G3_ref[...], preferred_element_type=jnp.float32)
                  + jnp.dot(x, AH_ref[...], preferred_element_type=jnp.float32)
                  + bf_ref[...])
    v_ref[...] = jnp.dot(d8 - p8, G3_ref[...],
                         preferred_element_type=jnp.float32)


def _tc1(h_p, pos8, A1, b1, A2, b2, G3, AH, bfr):
    full = lambda r, c: pl.BlockSpec((r, c), lambda i: (0, 0))
    blk = pl.BlockSpec((ROWB, None), lambda i: (i, 0))
    return pl.pallas_call(
        _tc1_body,
        grid=(NBLK,),
        in_specs=[pl.BlockSpec((ROWB, C), lambda i: (i, 0)),
                  pl.BlockSpec((ROWB, 8), lambda i: (i, 0)),
                  full(C, C), full(1, C), full(C, 8), full(1, 8),
                  full(8, C), full(C, C), full(1, C)],
        out_specs=[pl.BlockSpec((ROWB, C), lambda i: (i, 0)),
                   pl.BlockSpec((ROWB, C), lambda i: (i, 0))],
        out_shape=[jax.ShapeDtypeStruct((NPAD, C), jnp.float32),
                   jax.ShapeDtypeStruct((NPAD, C), jnp.float32)],
    )(h_p, pos8, A1, b1, A2, b2, G3, AH, bfr)


def _sc_body(u_hbm, src_hbm, dst_hbm, out_hbm,
             acc, dstb, srcb, sel, dlocb, rows, sem):
    wid = lax.axis_index("s") * 2 + lax.axis_index("c")
    lo = wid * NPW

    neg = jnp.full((16,), -jnp.inf, jnp.float32)

    def init_acc(r, _):
        for c8 in range(8):
            acc[r, pl.ds(c8 * 16, 16)] = neg
        return 0

    lax.fori_loop(0, NPW + 1, init_acc, 0)

    zero16 = jnp.zeros((16,), jnp.int32)

    def init_sel(g, _):
        sel[pl.ds(g * 16, 16)] = zero16
        dlocb[pl.ds(g * 16, 16)] = zero16
        return 0

    lax.fori_loop(0, (CH + 176) // 16, init_sel, 0)

    def chunk_body(ci, _):
        pltpu.sync_copy(dst_hbm.at[pl.ds(ci * CH, CH)], dstb)
        pltpu.sync_copy(src_hbm.at[pl.ds(ci * CH, CH)], srcb)

        zk = jnp.zeros((16,), jnp.int32)

        def scan_body(g, cnt):
            d = dstb[pl.ds(g * 16, 16)]
            s = srcb[pl.ds(g * 16, 16)]
            m = (d >= lo) & (d < lo + NPW)
            _, s_c, _ = plsc.sort_key_val(zk, s, mask=m)
            _, dl_c, _ = plsc.sort_key_val(zk, d - lo, mask=m)
            sel[pl.ds(cnt, 16)] = s_c
            dlocb[pl.ds(cnt, 16)] = dl_c
            pc = plsc.all_reduce_population_count(m)
            return cnt + pc[0]

        cnt = lax.fori_loop(0, NGR, scan_body, 0)
        dlocb[pl.ds(cnt, 16)] = jnp.full((16,), NPW, jnp.int32)
        cnt_pad = ((cnt + 15) // 16) * 16
        nb = (cnt_pad + GB - 1) // GB

        def batch_body(b, _):
            cp = pltpu.async_copy(u_hbm.at[sel.at[pl.ds(b * GB, GB)]],
                                  rows, sem)
            cp.wait()
            ng = (jnp.minimum((b + 1) * GB, cnt_pad) - b * GB) // 16

            def group_body(gj, _):
                base = gj * 16
                dl16 = dlocb[pl.ds(b * GB + base, 16)]
                for k in range(16):
                    dl = dl16[k]
                    r = base + k
                    for c8 in range(8):
                        sl = pl.ds(c8 * 16, 16)
                        acc[dl, sl] = jnp.maximum(acc[dl, sl], rows[r, sl])
                return 0

            lax.fori_loop(0, ng, group_body, 0)
            return 0

        lax.fori_loop(0, nb, batch_body, 0)
        return 0

    lax.fori_loop(0, E // CH, chunk_body, 0)
    pltpu.sync_copy(acc.at[pl.ds(0, NPW)], out_hbm.at[pl.ds(lo, NPW)])


def _sc_segmax(u, src, dst):
    mesh = plsc.VectorSubcoreMesh(core_axis_name="c", subcore_axis_name="s")
    f = pl.kernel(
        _sc_body,
        out_type=jax.ShapeDtypeStruct((NPAD, C), jnp.float32),
        mesh=mesh,
        scratch_types=[
            pltpu.VMEM((NPW + 1, C), jnp.float32),
            pltpu.VMEM((CH,), jnp.int32),
            pltpu.VMEM((CH,), jnp.int32),
            pltpu.VMEM((CH + 176,), jnp.int32),
            pltpu.VMEM((CH + 176,), jnp.int32),
            pltpu.VMEM((GB, C), jnp.float32),
            pltpu.SemaphoreType.DMA,
        ],
        compiler_params=pltpu.CompilerParams(needs_layout_passes=False),
    )
    return f(u, src, dst)


def _tc2a_body(smax_ref, v_ref, h_ref, noise_ref, G1_ref, c1_ref, G2_ref,
               c2_ref, ns_ref, hh_ref, sums_ref):
    i = pl.program_id(0)
    sm = smax_ref[...]
    agg = jnp.where(jnp.isneginf(sm), 0.0, _leaky(sm + v_ref[...]))
    t = _leaky(jnp.dot(agg, G1_ref[...], preferred_element_type=jnp.float32)
               + c1_ref[...])
    out = (jnp.dot(t, G2_ref[...], preferred_element_type=jnp.float32)
           + c2_ref[...])
    hh = _leaky(h_ref[...] + out + noise_ref[...] * ns_ref[0, 0])
    rows = i * ROWB + lax.broadcasted_iota(jnp.int32, (ROWB, 1), 0)
    hh = jnp.where(rows < N, hh, 0.0)
    hh_ref[...] = hh

    s1 = jnp.sum(hh, axis=0, keepdims=True)
    s2 = jnp.sum(hh * hh, axis=0, keepdims=True)

    @pl.when(i == 0)
    def _():
        sums_ref[...] = jnp.zeros_like(sums_ref)

    sums_ref[0:1, :] += s1
    sums_ref[1:2, :] += s2


def _tc2a(smax, v, h_p, noise_p, G1, c1, G2, c2, ns):
    full = lambda r, c: pl.BlockSpec((r, c), lambda i: (0, 0))
    rb = pl.BlockSpec((ROWB, C), lambda i: (i, 0))
    return pl.pallas_call(
        _tc2a_body,
        grid=(NBLK,),
        in_specs=[rb, rb, rb, rb, full(C, C), full(1, C), full(C, C),
                  full(1, C), full(1, 1)],
        out_specs=[rb, full(8, C)],
        out_shape=[jax.ShapeDtypeStruct((NPAD, C), jnp.float32),
                   jax.ShapeDtypeStruct((8, C), jnp.float32)],
    )(smax, v, h_p, noise_p, G1, c1, G2, c2, ns)


def _tc2b_body(hh_ref, sums_ref, style_ref, Wa_ref, ba_ref, o_ref):
    s1 = sums_ref[0:1, :]
    s2 = sums_ref[1:2, :]
    mean = s1 * (1.0 / N)
    var = s2 * (1.0 / N) - mean * mean
    inv = lax.rsqrt(var + 1e-5)
    st = (jnp.dot(style_ref[...], Wa_ref[...],
                  preferred_element_type=jnp.float32) + ba_ref[...])
    gamma = st[:, :C]
    beta = st[:, C:]
    o_ref[...] = gamma * ((hh_ref[...] - mean) * inv) + beta


def _tc2b(hh, sums, style_p, WaT, ba):
    full = lambda r, c: pl.BlockSpec((r, c), lambda i: (0, 0))
    rb = pl.BlockSpec((ROWB, C), lambda i: (i, 0))
    return pl.pallas_call(
        _tc2b_body,
        grid=(NBLK,),
        in_specs=[rb, full(8, C), rb, full(C, 2 * C), full(1, 2 * C)],
        out_specs=rb,
        out_shape=jax.ShapeDtypeStruct((NPAD, C), jnp.float32),
    )(hh, sums, style_p, WaT, ba)


@jax.jit
def kernel(h, pos, style, noise, W1h, b1h, W2h, b2h, Wf, bf, W1g, b1g, W2g,
           b2g, W_aff, b_aff, noise_strength, edge_index):
    pad = NPAD - N
    h_p = jnp.pad(h, ((0, pad), (0, 0)))
    pos8 = jnp.pad(pos, ((0, pad), (0, 5)))
    noise_p = jnp.pad(noise, ((0, pad), (0, 0)))
    style_p = jnp.pad(style, ((0, pad), (0, 0)))

    A1 = W1h.T
    b1 = b1h.reshape(1, C)
    A2 = jnp.pad(W2h.T, ((0, 0), (0, 5)))
    b2 = jnp.pad(b2h, (0, 5)).reshape(1, 8)
    G3 = jnp.pad(Wf[:, :3].T, ((0, 5), (0, 0)))
    AH = Wf[:, 3:].T
    bfr = bf.reshape(1, C)
    G1 = W1g.T
    c1 = b1g.reshape(1, C)
    G2 = W2g.T
    c2 = b2g.reshape(1, C)
    WaT = W_aff.T
    ba = b_aff.reshape(1, 2 * C)
    ns = noise_strength.reshape(1, 1)

    u, v = _tc1(h_p, pos8, A1, b1, A2, b2, G3, AH, bfr)
    smax = _sc_segmax(u, edge_index[0], edge_index[1])
    hh, sums = _tc2a(smax, v, h_p, noise_p, G1, c1, G2, c2, ns)
    final = _tc2b(hh, sums, style_p, WaT, ba)
    return final[:N]


# speedup vs baseline: 3.1130x; 3.1130x over previous
import jax
import jax.numpy as jnp
from jax import lax
from jax.experimental import pallas as pl
from jax.experimental.pallas import tpu as pltpu
from jax.experimental.pallas import tpu_sc as plsc

N = 10000
E = 320000
C = 128
NW = 32
CPW = C // NW
NPAD = 10240
NT = NPAD + 4
NTW = NT * CPW
CH = 2560
NGR = CH // 16
NCH = E // CH
ROWB = 1024
NBLK = NPAD // ROWB


def _leaky(x):
    return jnp.where(x >= 0, x, 0.01 * x)


def _tc1_body(h_ref, pos_ref, A1_ref, b1_ref, A2_ref, b2_ref, G3_ref, AH_ref,
              bf_ref, u_ref, v_ref):
    x = h_ref[...]
    p8 = pos_ref[...]
    t1 = _leaky(jnp.dot(x, A1_ref[...], preferred_element_type=jnp.float32)
                + b1_ref[...])
    d8 = jnp.tanh(jnp.dot(t1, A2_ref[...], preferred_element_type=jnp.float32)
                  + b2_ref[...])
    u_ref[...] = (jnp.dot(p8, G3_ref[...], preferred_element_type=jnp.float32)
                  + jnp.dot(x, AH_ref[...], preferred_element_type=jnp.float32)
                  + bf_ref[...])
    v_ref[...] = jnp.dot(d8 - p8, G3_ref[...],
                         preferred_element_type=jnp.float32)


def _tc1(h_p, pos8, A1, b1, A2, b2, G3, AH, bfr):
    full = lambda r, c: pl.BlockSpec((r, c), lambda i: (0, 0))
    return pl.pallas_call(
        _tc1_body,
        grid=(NBLK,),
        in_specs=[pl.BlockSpec((ROWB, C), lambda i: (i, 0)),
                  pl.BlockSpec((ROWB, 8), lambda i: (i, 0)),
                  full(C, C), full(1, C), full(C, 8), full(1, 8),
                  full(8, C), full(C, C), full(1, C)],
        out_specs=[pl.BlockSpec((ROWB, C), lambda i: (i, 0)),
                   pl.BlockSpec((ROWB, C), lambda i: (i, 0))],
        out_shape=[jax.ShapeDtypeStruct((NPAD, C), jnp.float32),
                   jax.ShapeDtypeStruct((NPAD, C), jnp.float32)],
    )(h_p, pos8, A1, b1, A2, b2, G3, AH, bfr)


def _sc_body(u_t, edge_hbm, out_hbm, uflat, aflat, eb, sems):
    wid = lax.axis_index("s") * 2 + lax.axis_index("c")

    pltpu.sync_copy(u_t.at[wid], uflat)

    neg = jnp.full((16,), -jnp.inf, jnp.float32)

    def init_acc(i, _):
        aflat[pl.ds(i * 16, 16)] = neg
        return 0

    lax.fori_loop(0, NTW // 16, init_acc, 0)

    pltpu.make_async_copy(edge_hbm.at[:, pl.ds(0, CH)], eb.at[0],
                          sems.at[0]).start()
    pltpu.make_async_copy(edge_hbm.at[:, pl.ds(CH, CH)], eb.at[1],
                          sems.at[1]).start()

    lane4 = lax.iota(jnp.int32, 16) < CPW

    def chunk_body(ci, _):
        p = ci % 2
        pltpu.make_async_copy(edge_hbm.at[:, pl.ds(ci * CH, CH)], eb.at[p],
                              sems.at[p]).wait()

        def group_body(g, _):
            so = eb[p, 0, pl.ds(g * 16, 16)] * CPW
            do = eb[p, 1, pl.ds(g * 16, 16)] * CPW
            for k in range(16):
                sa = so[k]
                da = do[k]
                uv = uflat[pl.ds(sa, 16)]
                av = aflat[pl.ds(da, 16)]
                aflat[pl.ds(da, 16)] = jnp.where(lane4,
                                                 jnp.maximum(av, uv), av)
            return 0

        lax.fori_loop(0, NGR, group_body, 0)

        @pl.when(ci + 2 < NCH)
        def _():
            pltpu.make_async_copy(edge_hbm.at[:, pl.ds((ci + 2) * CH, CH)],
                                  eb.at[p], sems.at[p]).start()

        return 0

    lax.fori_loop(0, NCH, chunk_body, 0)
    pltpu.sync_copy(aflat, out_hbm.at[wid])


def _sc_segmax(u_t, edge_index):
    mesh = plsc.VectorSubcoreMesh(core_axis_name="c", subcore_axis_name="s")
    f = pl.kernel(
        _sc_body,
        out_type=jax.ShapeDtypeStruct((NW, NTW), jnp.float32),
        mesh=mesh,
        scratch_types=[
            pltpu.VMEM((NTW,), jnp.float32),
            pltpu.VMEM((NTW,), jnp.float32),
            pltpu.VMEM((2, 2, CH), jnp.int32),
            pltpu.SemaphoreType.DMA((2,)),
        ],
        compiler_params=pltpu.CompilerParams(needs_layout_passes=False),
    )
    return f(u_t, edge_index)


def _tc2a_body(smax_ref, v_ref, h_ref, noise_ref, G1_ref, c1_ref, G2_ref,
               c2_ref, ns_ref, hh_ref, sums_ref):
    i = pl.program_id(0)
    sm = smax_ref[...]
    agg = jnp.where(jnp.isneginf(sm), 0.0, _leaky(sm + v_ref[...]))
    t = _leaky(jnp.dot(agg, G1_ref[...], preferred_element_type=jnp.float32)
               + c1_ref[...])
    out = (jnp.dot(t, G2_ref[...], preferred_element_type=jnp.float32)
           + c2_ref[...])
    hh = _leaky(h_ref[...] + out + noise_ref[...] * ns_ref[0, 0])
    rows = i * ROWB + lax.broadcasted_iota(jnp.int32, (ROWB, 1), 0)
    hh = jnp.where(rows < N, hh, 0.0)
    hh_ref[...] = hh

    s1 = jnp.sum(hh, axis=0, keepdims=True)
    s2 = jnp.sum(hh * hh, axis=0, keepdims=True)

    @pl.when(i == 0)
    def _():
        sums_ref[...] = jnp.zeros_like(sums_ref)

    sums_ref[0:1, :] += s1
    sums_ref[1:2, :] += s2


def _tc2a(smax, v, h_p, noise_p, G1, c1, G2, c2, ns):
    full = lambda r, c: pl.BlockSpec((r, c), lambda i: (0, 0))
    rb = pl.BlockSpec((ROWB, C), lambda i: (i, 0))
    return pl.pallas_call(
        _tc2a_body,
        grid=(NBLK,),
        in_specs=[rb, rb, rb, rb, full(C, C), full(1, C), full(C, C),
                  full(1, C), full(1, 1)],
        out_specs=[rb, full(8, C)],
        out_shape=[jax.ShapeDtypeStruct((NPAD, C), jnp.float32),
                   jax.ShapeDtypeStruct((8, C), jnp.float32)],
    )(smax, v, h_p, noise_p, G1, c1, G2, c2, ns)


def _tc2b_body(hh_ref, sums_ref, style_ref, Wa_ref, ba_ref, o_ref):
    s1 = sums_ref[0:1, :]
    s2 = sums_ref[1:2, :]
    mean = s1 * (1.0 / N)
    var = s2 * (1.0 / N) - mean * mean
    inv = lax.rsqrt(var + 1e-5)
    st = (jnp.dot(style_ref[...], Wa_ref[...],
                  preferred_element_type=jnp.float32) + ba_ref[...])
    gamma = st[:, :C]
    beta = st[:, C:]
    o_ref[...] = gamma * ((hh_ref[...] - mean) * inv) + beta


def _tc2b(hh, sums, style_p, WaT, ba):
    full = lambda r, c: pl.BlockSpec((r, c), lambda i: (0, 0))
    rb = pl.BlockSpec((ROWB, C), lambda i: (i, 0))
    return pl.pallas_call(
        _tc2b_body,
        grid=(NBLK,),
        in_specs=[rb, full(8, C), rb, full(C, 2 * C), full(1, 2 * C)],
        out_specs=rb,
        out_shape=jax.ShapeDtypeStruct((NPAD, C), jnp.float32),
    )(hh, sums, style_p, WaT, ba)


@jax.jit
def kernel(h, pos, style, noise, W1h, b1h, W2h, b2h, Wf, bf, W1g, b1g, W2g,
           b2g, W_aff, b_aff, noise_strength, edge_index):
    pad = NPAD - N
    h_p = jnp.pad(h, ((0, pad), (0, 0)))
    pos8 = jnp.pad(pos, ((0, pad), (0, 5)))
    noise_p = jnp.pad(noise, ((0, pad), (0, 0)))
    style_p = jnp.pad(style, ((0, pad), (0, 0)))

    A1 = W1h.T
    b1 = b1h.reshape(1, C)
    A2 = jnp.pad(W2h.T, ((0, 0), (0, 5)))
    b2 = jnp.pad(b2h, (0, 5)).reshape(1, 8)
    G3 = jnp.pad(Wf[:, :3].T, ((0, 5), (0, 0)))
    AH = Wf[:, 3:].T
    bfr = bf.reshape(1, C)
    G1 = W1g.T
    c1 = b1g.reshape(1, C)
    G2 = W2g.T
    c2 = b2g.reshape(1, C)
    WaT = W_aff.T
    ba = b_aff.reshape(1, 2 * C)
    ns = noise_strength.reshape(1, 1)

    u, v = _tc1(h_p, pos8, A1, b1, A2, b2, G3, AH, bfr)

    u_t = (jnp.pad(u, ((0, NT - NPAD), (0, 0)))
           .reshape(NT, NW, CPW).transpose(1, 0, 2).reshape(NW, NTW))
    smax_t = _sc_segmax(u_t, edge_index)
    smax = (smax_t.reshape(NW, NT, CPW).transpose(1, 0, 2)
            .reshape(NT, C)[:NPAD])

    hh, sums = _tc2a(smax, v, h_p, noise_p, G1, c1, G2, c2, ns)
    final = _tc2b(hh, sums, style_p, WaT, ba)
    return final[:N]


# speedup vs baseline: 3.3675x; 1.0817x over previous
import jax
import jax.numpy as jnp
from jax import lax
from jax.experimental import pallas as pl
from jax.experimental.pallas import tpu as pltpu
from jax.experimental.pallas import tpu_sc as plsc

N = 10000
E = 320000
C = 128
NW = 32
CPW = C // NW
NPAD = 10240
NT = NPAD + 4
NTW = NT * CPW
CH = 1280
NGR = CH // 16
NCH = E // CH
ROWB = 1024
NBLK = NPAD // ROWB


def _leaky(x):
    return jnp.where(x >= 0, x, 0.01 * x)


def _tc1_body(h_ref, pos_ref, A1_ref, b1_ref, A2_ref, b2_ref, G3_ref, AH_ref,
              bf_ref, u_ref, v_ref):
    x = h_ref[...]
    p8 = pos_ref[...]
    t1 = _leaky(jnp.dot(x, A1_ref[...], preferred_element_type=jnp.float32)
                + b1_ref[...])
    d8 = jnp.tanh(jnp.dot(t1, A2_ref[...], preferred_element_type=jnp.float32)
                  + b2_ref[...])
    u_ref[...] = (jnp.dot(p8, G3_ref[...], preferred_element_type=jnp.float32)
                  + jnp.dot(x, AH_ref[...], preferred_element_type=jnp.float32)
                  + bf_ref[...])
    v_ref[...] = jnp.dot(d8 - p8, G3_ref[...],
                         preferred_element_type=jnp.float32)


def _tc1(h_p, pos8, A1, b1, A2, b2, G3, AH, bfr):
    full = lambda r, c: pl.BlockSpec((r, c), lambda i: (0, 0))
    return pl.pallas_call(
        _tc1_body,
        grid=(NBLK,),
        in_specs=[pl.BlockSpec((ROWB, C), lambda i: (i, 0)),
                  pl.BlockSpec((ROWB, 8), lambda i: (i, 0)),
                  full(C, C), full(1, C), full(C, 8), full(1, 8),
                  full(8, C), full(C, C), full(1, C)],
        out_specs=[pl.BlockSpec((ROWB, C), lambda i: (i, 0)),
                   pl.BlockSpec((ROWB, C), lambda i: (i, 0))],
        out_shape=[jax.ShapeDtypeStruct((NPAD, C), jnp.float32),
                   jax.ShapeDtypeStruct((NPAD, C), jnp.float32)],
    )(h_p, pos8, A1, b1, A2, b2, G3, AH, bfr)


def _sc_body(u_t, edge_hbm, out_hbm, uflat, acc_a, acc_b, eb, sems):
    wid = lax.axis_index("s") * 2 + lax.axis_index("c")

    pltpu.sync_copy(u_t.at[wid], uflat)

    neg = jnp.full((16,), -jnp.inf, jnp.float32)

    def init_acc(i, _):
        acc_a[pl.ds(i * 16, 16)] = neg
        acc_b[pl.ds(i * 16, 16)] = neg
        return 0

    lax.fori_loop(0, NTW // 16, init_acc, 0)

    pltpu.make_async_copy(edge_hbm.at[:, pl.ds(0, CH)], eb.at[0],
                          sems.at[0]).start()
    pltpu.make_async_copy(edge_hbm.at[:, pl.ds(CH, CH)], eb.at[1],
                          sems.at[1]).start()

    lane4 = lax.iota(jnp.int32, 16) < CPW

    def chunk_body(ci, _):
        p = ci % 2
        pltpu.make_async_copy(edge_hbm.at[:, pl.ds(ci * CH, CH)], eb.at[p],
                              sems.at[p]).wait()

        def group_body(g, _):
            so = eb[p, 0, pl.ds(g * 16, 16)] * CPW
            do = eb[p, 1, pl.ds(g * 16, 16)] * CPW
            for k in range(16):
                acc = acc_a if k % 2 == 0 else acc_b
                sa = so[k]
                da = do[k]
                uv = jnp.where(lane4, uflat[pl.ds(sa, 16)], neg)
                acc[pl.ds(da, 16)] = jnp.maximum(acc[pl.ds(da, 16)], uv)
            return 0

        lax.fori_loop(0, NGR, group_body, 0)

        @pl.when(ci + 2 < NCH)
        def _():
            pltpu.make_async_copy(edge_hbm.at[:, pl.ds((ci + 2) * CH, CH)],
                                  eb.at[p], sems.at[p]).start()

        return 0

    lax.fori_loop(0, NCH, chunk_body, 0)

    def merge(i, _):
        sl = pl.ds(i * 16, 16)
        acc_a[sl] = jnp.maximum(acc_a[sl], acc_b[sl])
        return 0

    lax.fori_loop(0, NTW // 16, merge, 0)
    pltpu.sync_copy(acc_a, out_hbm.at[wid])


def _sc_segmax(u_t, edge_index):
    mesh = plsc.VectorSubcoreMesh(core_axis_name="c", subcore_axis_name="s")
    f = pl.kernel(
        _sc_body,
        out_type=jax.ShapeDtypeStruct((NW, NTW), jnp.float32),
        mesh=mesh,
        scratch_types=[
            pltpu.VMEM((NTW,), jnp.float32),
            pltpu.VMEM((NTW,), jnp.float32),
            pltpu.VMEM((NTW,), jnp.float32),
            pltpu.VMEM((2, 2, CH), jnp.int32),
            pltpu.SemaphoreType.DMA((2,)),
        ],
        compiler_params=pltpu.CompilerParams(needs_layout_passes=False),
    )
    return f(u_t, edge_index)


def _tc2a_body(smax_ref, v_ref, h_ref, noise_ref, G1_ref, c1_ref, G2_ref,
               c2_ref, ns_ref, hh_ref, sums_ref):
    i = pl.program_id(0)
    sm = smax_ref[...]
    agg = jnp.where(jnp.isneginf(sm), 0.0, _leaky(sm + v_ref[...]))
    t = _leaky(jnp.dot(agg, G1_ref[...], preferred_element_type=jnp.float32)
               + c1_ref[...])
    out = (jnp.dot(t, G2_ref[...], preferred_element_type=jnp.float32)
           + c2_ref[...])
    hh = _leaky(h_ref[...] + out + noise_ref[...] * ns_ref[0, 0])
    rows = i * ROWB + lax.broadcasted_iota(jnp.int32, (ROWB, 1), 0)
    hh = jnp.where(rows < N, hh, 0.0)
    hh_ref[...] = hh

    s1 = jnp.sum(hh, axis=0, keepdims=True)
    s2 = jnp.sum(hh * hh, axis=0, keepdims=True)

    @pl.when(i == 0)
    def _():
        sums_ref[...] = jnp.zeros_like(sums_ref)

    sums_ref[0:1, :] += s1
    sums_ref[1:2, :] += s2


def _tc2a(smax, v, h_p, noise_p, G1, c1, G2, c2, ns):
    full = lambda r, c: pl.BlockSpec((r, c), lambda i: (0, 0))
    rb = pl.BlockSpec((ROWB, C), lambda i: (i, 0))
    return pl.pallas_call(
        _tc2a_body,
        grid=(NBLK,),
        in_specs=[rb, rb, rb, rb, full(C, C), full(1, C), full(C, C),
                  full(1, C), full(1, 1)],
        out_specs=[rb, full(8, C)],
        out_shape=[jax.ShapeDtypeStruct((NPAD, C), jnp.float32),
                   jax.ShapeDtypeStruct((8, C), jnp.float32)],
    )(smax, v, h_p, noise_p, G1, c1, G2, c2, ns)


def _tc2b_body(hh_ref, sums_ref, style_ref, Wa_ref, ba_ref, o_ref):
    s1 = sums_ref[0:1, :]
    s2 = sums_ref[1:2, :]
    mean = s1 * (1.0 / N)
    var = s2 * (1.0 / N) - mean * mean
    inv = lax.rsqrt(var + 1e-5)
    st = (jnp.dot(style_ref[...], Wa_ref[...],
                  preferred_element_type=jnp.float32) + ba_ref[...])
    gamma = st[:, :C]
    beta = st[:, C:]
    o_ref[...] = gamma * ((hh_ref[...] - mean) * inv) + beta


def _tc2b(hh, sums, style_p, WaT, ba):
    full = lambda r, c: pl.BlockSpec((r, c), lambda i: (0, 0))
    rb = pl.BlockSpec((ROWB, C), lambda i: (i, 0))
    return pl.pallas_call(
        _tc2b_body,
        grid=(NBLK,),
        in_specs=[rb, full(8, C), rb, full(C, 2 * C), full(1, 2 * C)],
        out_specs=rb,
        out_shape=jax.ShapeDtypeStruct((NPAD, C), jnp.float32),
    )(hh, sums, style_p, WaT, ba)


@jax.jit
def kernel(h, pos, style, noise, W1h, b1h, W2h, b2h, Wf, bf, W1g, b1g, W2g,
           b2g, W_aff, b_aff, noise_strength, edge_index):
    pad = NPAD - N
    h_p = jnp.pad(h, ((0, pad), (0, 0)))
    pos8 = jnp.pad(pos, ((0, pad), (0, 5)))
    noise_p = jnp.pad(noise, ((0, pad), (0, 0)))
    style_p = jnp.pad(style, ((0, pad), (0, 0)))

    A1 = W1h.T
    b1 = b1h.reshape(1, C)
    A2 = jnp.pad(W2h.T, ((0, 0), (0, 5)))
    b2 = jnp.pad(b2h, (0, 5)).reshape(1, 8)
    G3 = jnp.pad(Wf[:, :3].T, ((0, 5), (0, 0)))
    AH = Wf[:, 3:].T
    bfr = bf.reshape(1, C)
    G1 = W1g.T
    c1 = b1g.reshape(1, C)
    G2 = W2g.T
    c2 = b2g.reshape(1, C)
    WaT = W_aff.T
    ba = b_aff.reshape(1, 2 * C)
    ns = noise_strength.reshape(1, 1)

    u, v = _tc1(h_p, pos8, A1, b1, A2, b2, G3, AH, bfr)

    u_t = (jnp.pad(u, ((0, NT - NPAD), (0, 0)))
           .reshape(NT, NW, CPW).transpose(1, 0, 2).reshape(NW, NTW))
    smax_t = _sc_segmax(u_t, edge_index)
    smax = (smax_t.reshape(NW, NT, CPW).transpose(1, 0, 2)
            .reshape(NT, C)[:NPAD])

    hh, sums = _tc2a(smax, v, h_p, noise_p, G1, c1, G2, c2, ns)
    final = _tc2b(hh, sums, style_p, WaT, ba)
    return final[:N]


# speedup vs baseline: 3.5423x; 1.0519x over previous
import jax
import jax.numpy as jnp
from jax import lax
from jax.experimental import pallas as pl
from jax.experimental.pallas import tpu as pltpu
from jax.experimental.pallas import tpu_sc as plsc

N = 10000
E = 320000
C = 128
NW = 32
CPW = C // NW
NPAD = 10240
NT = NPAD + 4
NTW = NT * CPW
CH = 2560
NGR = CH // 16
NCH = E // CH
ROWB = 1024
NBLK = NPAD // ROWB


def _leaky(x):
    return jnp.where(x >= 0, x, 0.01 * x)


def _tc1_body(h_ref, pos_ref, A1_ref, b1_ref, A2_ref, b2_ref, G3_ref, AH_ref,
              bf_ref, u_ref, v_ref):
    x = h_ref[...]
    p8 = pos_ref[...]
    t1 = _leaky(jnp.dot(x, A1_ref[...], preferred_element_type=jnp.float32)
                + b1_ref[...])
    d8 = jnp.tanh(jnp.dot(t1, A2_ref[...], preferred_element_type=jnp.float32)
                  + b2_ref[...])
    u_ref[...] = (jnp.dot(p8, G3_ref[...], preferred_element_type=jnp.float32)
                  + jnp.dot(x, AH_ref[...], preferred_element_type=jnp.float32)
                  + bf_ref[...])
    v_ref[...] = jnp.dot(d8 - p8, G3_ref[...],
                         preferred_element_type=jnp.float32)


def _tc1(h_p, pos8, A1, b1, A2, b2, G3, AH, bfr):
    full = lambda r, c: pl.BlockSpec((r, c), lambda i: (0, 0))
    return pl.pallas_call(
        _tc1_body,
        grid=(NBLK,),
        in_specs=[pl.BlockSpec((ROWB, C), lambda i: (i, 0)),
                  pl.BlockSpec((ROWB, 8), lambda i: (i, 0)),
                  full(C, C), full(1, C), full(C, 8), full(1, 8),
                  full(8, C), full(C, C), full(1, C)],
        out_specs=[pl.BlockSpec((ROWB, C), lambda i: (i, 0)),
                   pl.BlockSpec((ROWB, C), lambda i: (i, 0))],
        out_shape=[jax.ShapeDtypeStruct((NPAD, C), jnp.float32),
                   jax.ShapeDtypeStruct((NPAD, C), jnp.float32)],
    )(h_p, pos8, A1, b1, A2, b2, G3, AH, bfr)


_PERM_DN = lax.GatherDimensionNumbers(offset_dims=(), collapsed_slice_dims=(0,),
                                      start_index_map=(0,))


def _perm(x, idx):
    return lax.gather(x, idx[:, None], _PERM_DN, slice_sizes=(1,),
                      mode=lax.GatherScatterMode.PROMISE_IN_BOUNDS)


def _sc_body(u_t, edge_hbm, out_hbm, uflat, aflat, eb, sems):
    wid = lax.axis_index("s") * 2 + lax.axis_index("c")

    pltpu.sync_copy(u_t.at[wid], uflat)

    neg = jnp.full((16,), -jnp.inf, jnp.float32)

    def init_acc(i, _):
        aflat[pl.ds(i * 16, 16)] = neg
        return 0

    lax.fori_loop(0, NTW // 16, init_acc, 0)

    pltpu.make_async_copy(edge_hbm.at[:, pl.ds(0, CH)], eb.at[0],
                          sems.at[0]).start()
    pltpu.make_async_copy(edge_hbm.at[:, pl.ds(CH, CH)], eb.at[1],
                          sems.at[1]).start()

    lanes = lax.iota(jnp.int32, 16)
    sh_idx = [jnp.maximum(lanes - (1 << b), 0) for b in range(4)]
    nxt_idx = jnp.minimum(lanes + 1, 15)
    is_last = lanes == 15

    def chunk_body(ci, _):
        p = ci % 2
        pltpu.make_async_copy(edge_hbm.at[:, pl.ds(ci * CH, CH)], eb.at[p],
                              sems.at[p]).wait()

        def group_body(g, _):
            s16 = eb[p, 0, pl.ds(g * 16, 16)]
            d16 = eb[p, 1, pl.ds(g * 16, 16)]
            d_s, pi = plsc.sort_key_val(d16, lanes)
            so_s = _perm(s16, pi) * CPW
            do_s = d_s * CPW
            ok = [d_s == _perm(d_s, ix) for ix in sh_idx]
            seg_end = is_last | (d_s != _perm(d_s, nxt_idx))
            for c in range(CPW):
                uv = plsc.load_gather(uflat, [so_s + c])
                for b in range(4):
                    uv = jnp.where(ok[b],
                                   jnp.maximum(uv, _perm(uv, sh_idx[b])), uv)
                av = plsc.load_gather(aflat, [do_s + c])
                plsc.store_scatter(aflat, [do_s + c], jnp.maximum(av, uv),
                                   mask=seg_end)
            return 0

        lax.fori_loop(0, NGR, group_body, 0)

        @pl.when(ci + 2 < NCH)
        def _():
            pltpu.make_async_copy(edge_hbm.at[:, pl.ds((ci + 2) * CH, CH)],
                                  eb.at[p], sems.at[p]).start()

        return 0

    lax.fori_loop(0, NCH, chunk_body, 0)
    pltpu.sync_copy(aflat, out_hbm.at[wid])


def _sc_segmax(u_t, edge_index):
    mesh = plsc.VectorSubcoreMesh(core_axis_name="c", subcore_axis_name="s")
    f = pl.kernel(
        _sc_body,
        out_type=jax.ShapeDtypeStruct((NW, NTW), jnp.float32),
        mesh=mesh,
        scratch_types=[
            pltpu.VMEM((NTW,), jnp.float32),
            pltpu.VMEM((NTW,), jnp.float32),
            pltpu.VMEM((2, 2, CH), jnp.int32),
            pltpu.SemaphoreType.DMA((2,)),
        ],
        compiler_params=pltpu.CompilerParams(needs_layout_passes=False),
    )
    return f(u_t, edge_index)


def _tc2a_body(smax_ref, v_ref, h_ref, noise_ref, G1_ref, c1_ref, G2_ref,
               c2_ref, ns_ref, hh_ref, sums_ref):
    i = pl.program_id(0)
    sm = smax_ref[...]
    agg = jnp.where(jnp.isneginf(sm), 0.0, _leaky(sm + v_ref[...]))
    t = _leaky(jnp.dot(agg, G1_ref[...], preferred_element_type=jnp.float32)
               + c1_ref[...])
    out = (jnp.dot(t, G2_ref[...], preferred_element_type=jnp.float32)
           + c2_ref[...])
    hh = _leaky(h_ref[...] + out + noise_ref[...] * ns_ref[0, 0])
    rows = i * ROWB + lax.broadcasted_iota(jnp.int32, (ROWB, 1), 0)
    hh = jnp.where(rows < N, hh, 0.0)
    hh_ref[...] = hh

    s1 = jnp.sum(hh, axis=0, keepdims=True)
    s2 = jnp.sum(hh * hh, axis=0, keepdims=True)

    @pl.when(i == 0)
    def _():
        sums_ref[...] = jnp.zeros_like(sums_ref)

    sums_ref[0:1, :] += s1
    sums_ref[1:2, :] += s2


def _tc2a(smax, v, h_p, noise_p, G1, c1, G2, c2, ns):
    full = lambda r, c: pl.BlockSpec((r, c), lambda i: (0, 0))
    rb = pl.BlockSpec((ROWB, C), lambda i: (i, 0))
    return pl.pallas_call(
        _tc2a_body,
        grid=(NBLK,),
        in_specs=[rb, rb, rb, rb, full(C, C), full(1, C), full(C, C),
                  full(1, C), full(1, 1)],
        out_specs=[rb, full(8, C)],
        out_shape=[jax.ShapeDtypeStruct((NPAD, C), jnp.float32),
                   jax.ShapeDtypeStruct((8, C), jnp.float32)],
    )(smax, v, h_p, noise_p, G1, c1, G2, c2, ns)


def _tc2b_body(hh_ref, sums_ref, style_ref, Wa_ref, ba_ref, o_ref):
    s1 = sums_ref[0:1, :]
    s2 = sums_ref[1:2, :]
    mean = s1 * (1.0 / N)
    var = s2 * (1.0 / N) - mean * mean
    inv = lax.rsqrt(var + 1e-5)
    st = (jnp.dot(style_ref[...], Wa_ref[...],
                  preferred_element_type=jnp.float32) + ba_ref[...])
    gamma = st[:, :C]
    beta = st[:, C:]
    o_ref[...] = gamma * ((hh_ref[...] - mean) * inv) + beta


def _tc2b(hh, sums, style_p, WaT, ba):
    full = lambda r, c: pl.BlockSpec((r, c), lambda i: (0, 0))
    rb = pl.BlockSpec((ROWB, C), lambda i: (i, 0))
    return pl.pallas_call(
        _tc2b_body,
        grid=(NBLK,),
        in_specs=[rb, full(8, C), rb, full(C, 2 * C), full(1, 2 * C)],
        out_specs=rb,
        out_shape=jax.ShapeDtypeStruct((NPAD, C), jnp.float32),
    )(hh, sums, style_p, WaT, ba)


@jax.jit
def kernel(h, pos, style, noise, W1h, b1h, W2h, b2h, Wf, bf, W1g, b1g, W2g,
           b2g, W_aff, b_aff, noise_strength, edge_index):
    pad = NPAD - N
    h_p = jnp.pad(h, ((0, pad), (0, 0)))
    pos8 = jnp.pad(pos, ((0, pad), (0, 5)))
    noise_p = jnp.pad(noise, ((0, pad), (0, 0)))
    style_p = jnp.pad(style, ((0, pad), (0, 0)))

    A1 = W1h.T
    b1 = b1h.reshape(1, C)
    A2 = jnp.pad(W2h.T, ((0, 0), (0, 5)))
    b2 = jnp.pad(b2h, (0, 5)).reshape(1, 8)
    G3 = jnp.pad(Wf[:, :3].T, ((0, 5), (0, 0)))
    AH = Wf[:, 3:].T
    bfr = bf.reshape(1, C)
    G1 = W1g.T
    c1 = b1g.reshape(1, C)
    G2 = W2g.T
    c2 = b2g.reshape(1, C)
    WaT = W_aff.T
    ba = b_aff.reshape(1, 2 * C)
    ns = noise_strength.reshape(1, 1)

    u, v = _tc1(h_p, pos8, A1, b1, A2, b2, G3, AH, bfr)

    u_t = (jnp.pad(u, ((0, NT - NPAD), (0, 0)))
           .reshape(NT, NW, CPW).transpose(1, 0, 2).reshape(NW, NTW))
    smax_t = _sc_segmax(u_t, edge_index)
    smax = (smax_t.reshape(NW, NT, CPW).transpose(1, 0, 2)
            .reshape(NT, C)[:NPAD])

    hh, sums = _tc2a(smax, v, h_p, noise_p, G1, c1, G2, c2, ns)
    final = _tc2b(hh, sums, style_p, WaT, ba)
    return final[:N]


# speedup vs baseline: 4.1717x; 1.1777x over previous
import jax
import jax.numpy as jnp
from jax import lax
from jax.experimental import pallas as pl
from jax.experimental.pallas import tpu as pltpu
from jax.experimental.pallas import tpu_sc as plsc

N = 10000
E = 320000
C = 128
NW = 32
CPW = C // NW
NPAD = 10240
NT = NPAD + 4
NTW = NT * CPW
CH = 2560
NGR = CH // 16
NCH = E // CH
ROWB = 1024
NBLK = NPAD // ROWB


def _leaky(x):
    return jnp.where(x >= 0, x, 0.01 * x)


def _tc1_body(h_ref, pos_ref, A1_ref, b1_ref, A2_ref, b2_ref, G3_ref, AH_ref,
              bf_ref, u_ref, v_ref):
    x = h_ref[...]
    p8 = pos_ref[...]
    t1 = _leaky(jnp.dot(x, A1_ref[...], preferred_element_type=jnp.float32)
                + b1_ref[...])
    d8 = jnp.tanh(jnp.dot(t1, A2_ref[...], preferred_element_type=jnp.float32)
                  + b2_ref[...])
    u_ref[...] = (jnp.dot(p8, G3_ref[...], preferred_element_type=jnp.float32)
                  + jnp.dot(x, AH_ref[...], preferred_element_type=jnp.float32)
                  + bf_ref[...])
    v_ref[...] = jnp.dot(d8 - p8, G3_ref[...],
                         preferred_element_type=jnp.float32)


def _tc1(h_p, pos8, A1, b1, A2, b2, G3, AH, bfr):
    full = lambda r, c: pl.BlockSpec((r, c), lambda i: (0, 0))
    return pl.pallas_call(
        _tc1_body,
        grid=(NBLK,),
        in_specs=[pl.BlockSpec((ROWB, C), lambda i: (i, 0)),
                  pl.BlockSpec((ROWB, 8), lambda i: (i, 0)),
                  full(C, C), full(1, C), full(C, 8), full(1, 8),
                  full(8, C), full(C, C), full(1, C)],
        out_specs=[pl.BlockSpec((ROWB, C), lambda i: (i, 0)),
                   pl.BlockSpec((ROWB, C), lambda i: (i, 0))],
        out_shape=[jax.ShapeDtypeStruct((NPAD, C), jnp.float32),
                   jax.ShapeDtypeStruct((NPAD, C), jnp.float32)],
    )(h_p, pos8, A1, b1, A2, b2, G3, AH, bfr)


_PERM_DN = lax.GatherDimensionNumbers(offset_dims=(), collapsed_slice_dims=(0,),
                                      start_index_map=(0,))


def _perm(x, idx):
    return lax.gather(x, idx[:, None], _PERM_DN, slice_sizes=(1,),
                      mode=lax.GatherScatterMode.PROMISE_IN_BOUNDS)


def _sc_body(u_t, edge_hbm, out_hbm, uflat, aflat, eb, sems):
    wid = lax.axis_index("s") * 2 + lax.axis_index("c")

    pltpu.sync_copy(u_t.at[wid], uflat)

    neg = jnp.full((16,), -jnp.inf, jnp.float32)

    def init_acc(i, _):
        aflat[pl.ds(i * 16, 16)] = neg
        return 0

    lax.fori_loop(0, NTW // 16, init_acc, 0)

    pltpu.make_async_copy(edge_hbm.at[:, pl.ds(0, CH)], eb.at[0],
                          sems.at[0]).start()
    pltpu.make_async_copy(edge_hbm.at[:, pl.ds(CH, CH)], eb.at[1],
                          sems.at[1]).start()

    lanes = lax.iota(jnp.int32, 16)
    sh_idx = [jnp.maximum(lanes - (1 << b), 0) for b in range(4)]
    nxt_idx = jnp.minimum(lanes + 1, 15)
    is_last = lanes == 15

    def chunk_body(ci, _):
        p = ci % 2
        pltpu.make_async_copy(edge_hbm.at[:, pl.ds(ci * CH, CH)], eb.at[p],
                              sems.at[p]).wait()

        def group_body(g, _):
            s16 = eb[p, 0, pl.ds(g * 16, 16)]
            d16 = eb[p, 1, pl.ds(g * 16, 16)]
            so = s16 * CPW
            do = d16 * CPW
            uvs = [plsc.load_gather(uflat, [so + c]) for c in range(CPW)]
            idxs = [do + c for c in range(CPW)]

            def rmw_round(_cnt):
                stills = []
                for c in range(CPW):
                    av = plsc.load_gather(aflat, [idxs[c]])
                    need = uvs[c] > av
                    plsc.store_scatter(aflat, [idxs[c]], uvs[c], mask=need)
                    av2 = plsc.load_gather(aflat, [idxs[c]])
                    stills.append(uvs[c] > av2)
                m = (stills[0] | stills[1]) | (stills[2] | stills[3])
                pc = plsc.all_reduce_population_count(m)
                return pc[0]

            lax.while_loop(lambda cnt: cnt > 0, rmw_round, jnp.int32(1))
            return 0

        lax.fori_loop(0, NGR, group_body, 0)

        @pl.when(ci + 2 < NCH)
        def _():
            pltpu.make_async_copy(edge_hbm.at[:, pl.ds((ci + 2) * CH, CH)],
                                  eb.at[p], sems.at[p]).start()

        return 0

    lax.fori_loop(0, NCH, chunk_body, 0)
    pltpu.sync_copy(aflat, out_hbm.at[wid])


def _sc_segmax(u_t, edge_index):
    mesh = plsc.VectorSubcoreMesh(core_axis_name="c", subcore_axis_name="s")
    f = pl.kernel(
        _sc_body,
        out_type=jax.ShapeDtypeStruct((NW, NTW), jnp.float32),
        mesh=mesh,
        scratch_types=[
            pltpu.VMEM((NTW,), jnp.float32),
            pltpu.VMEM((NTW,), jnp.float32),
            pltpu.VMEM((2, 2, CH), jnp.int32),
            pltpu.SemaphoreType.DMA((2,)),
        ],
        compiler_params=pltpu.CompilerParams(needs_layout_passes=False),
    )
    return f(u_t, edge_index)


def _tc2a_body(smax_ref, v_ref, h_ref, noise_ref, G1_ref, c1_ref, G2_ref,
               c2_ref, ns_ref, hh_ref, sums_ref):
    i = pl.program_id(0)
    sm = smax_ref[...]
    agg = jnp.where(jnp.isneginf(sm), 0.0, _leaky(sm + v_ref[...]))
    t = _leaky(jnp.dot(agg, G1_ref[...], preferred_element_type=jnp.float32)
               + c1_ref[...])
    out = (jnp.dot(t, G2_ref[...], preferred_element_type=jnp.float32)
           + c2_ref[...])
    hh = _leaky(h_ref[...] + out + noise_ref[...] * ns_ref[0, 0])
    rows = i * ROWB + lax.broadcasted_iota(jnp.int32, (ROWB, 1), 0)
    hh = jnp.where(rows < N, hh, 0.0)
    hh_ref[...] = hh

    s1 = jnp.sum(hh, axis=0, keepdims=True)
    s2 = jnp.sum(hh * hh, axis=0, keepdims=True)

    @pl.when(i == 0)
    def _():
        sums_ref[...] = jnp.zeros_like(sums_ref)

    sums_ref[0:1, :] += s1
    sums_ref[1:2, :] += s2


def _tc2a(smax, v, h_p, noise_p, G1, c1, G2, c2, ns):
    full = lambda r, c: pl.BlockSpec((r, c), lambda i: (0, 0))
    rb = pl.BlockSpec((ROWB, C), lambda i: (i, 0))
    return pl.pallas_call(
        _tc2a_body,
        grid=(NBLK,),
        in_specs=[rb, rb, rb, rb, full(C, C), full(1, C), full(C, C),
                  full(1, C), full(1, 1)],
        out_specs=[rb, full(8, C)],
        out_shape=[jax.ShapeDtypeStruct((NPAD, C), jnp.float32),
                   jax.ShapeDtypeStruct((8, C), jnp.float32)],
    )(smax, v, h_p, noise_p, G1, c1, G2, c2, ns)


def _tc2b_body(hh_ref, sums_ref, style_ref, Wa_ref, ba_ref, o_ref):
    s1 = sums_ref[0:1, :]
    s2 = sums_ref[1:2, :]
    mean = s1 * (1.0 / N)
    var = s2 * (1.0 / N) - mean * mean
    inv = lax.rsqrt(var + 1e-5)
    st = (jnp.dot(style_ref[...], Wa_ref[...],
                  preferred_element_type=jnp.float32) + ba_ref[...])
    gamma = st[:, :C]
    beta = st[:, C:]
    o_ref[...] = gamma * ((hh_ref[...] - mean) * inv) + beta


def _tc2b(hh, sums, style_p, WaT, ba):
    full = lambda r, c: pl.BlockSpec((r, c), lambda i: (0, 0))
    rb = pl.BlockSpec((ROWB, C), lambda i: (i, 0))
    return pl.pallas_call(
        _tc2b_body,
        grid=(NBLK,),
        in_specs=[rb, full(8, C), rb, full(C, 2 * C), full(1, 2 * C)],
        out_specs=rb,
        out_shape=jax.ShapeDtypeStruct((NPAD, C), jnp.float32),
    )(hh, sums, style_p, WaT, ba)


@jax.jit
def kernel(h, pos, style, noise, W1h, b1h, W2h, b2h, Wf, bf, W1g, b1g, W2g,
           b2g, W_aff, b_aff, noise_strength, edge_index):
    pad = NPAD - N
    h_p = jnp.pad(h, ((0, pad), (0, 0)))
    pos8 = jnp.pad(pos, ((0, pad), (0, 5)))
    noise_p = jnp.pad(noise, ((0, pad), (0, 0)))
    style_p = jnp.pad(style, ((0, pad), (0, 0)))

    A1 = W1h.T
    b1 = b1h.reshape(1, C)
    A2 = jnp.pad(W2h.T, ((0, 0), (0, 5)))
    b2 = jnp.pad(b2h, (0, 5)).reshape(1, 8)
    G3 = jnp.pad(Wf[:, :3].T, ((0, 5), (0, 0)))
    AH = Wf[:, 3:].T
    bfr = bf.reshape(1, C)
    G1 = W1g.T
    c1 = b1g.reshape(1, C)
    G2 = W2g.T
    c2 = b2g.reshape(1, C)
    WaT = W_aff.T
    ba = b_aff.reshape(1, 2 * C)
    ns = noise_strength.reshape(1, 1)

    u, v = _tc1(h_p, pos8, A1, b1, A2, b2, G3, AH, bfr)

    u_t = (jnp.pad(u, ((0, NT - NPAD), (0, 0)))
           .reshape(NT, NW, CPW).transpose(1, 0, 2).reshape(NW, NTW))
    smax_t = _sc_segmax(u_t, edge_index)
    smax = (smax_t.reshape(NW, NT, CPW).transpose(1, 0, 2)
            .reshape(NT, C)[:NPAD])

    hh, sums = _tc2a(smax, v, h_p, noise_p, G1, c1, G2, c2, ns)
    final = _tc2b(hh, sums, style_p, WaT, ba)
    return final[:N]


# speedup vs baseline: 5.1848x; 1.2429x over previous
import jax
import jax.numpy as jnp
from jax import lax
from jax.experimental import pallas as pl
from jax.experimental.pallas import tpu as pltpu
from jax.experimental.pallas import tpu_sc as plsc

N = 10000
E = 320000
C = 128
NW = 32
CPW = C // NW
NPAD = 10240
NT = NPAD + 4
NTW = NT * CPW
CH = 1280
NGR = CH // 16
NCH = E // CH
GPB = 8
NBL = NGR // GPB
ROWB = 1024
NBLK = NPAD // ROWB


def _leaky(x):
    return jnp.where(x >= 0, x, 0.01 * x)


def _tc1_body(h_ref, pos_ref, A1_ref, b1_ref, A2_ref, b2_ref, G3_ref, AH_ref,
              bf_ref, u_ref, v_ref):
    x = h_ref[...]
    p8 = pos_ref[...]
    t1 = _leaky(jnp.dot(x, A1_ref[...], preferred_element_type=jnp.float32)
                + b1_ref[...])
    d8 = jnp.tanh(jnp.dot(t1, A2_ref[...], preferred_element_type=jnp.float32)
                  + b2_ref[...])
    u_ref[...] = (jnp.dot(p8, G3_ref[...], preferred_element_type=jnp.float32)
                  + jnp.dot(x, AH_ref[...], preferred_element_type=jnp.float32)
                  + bf_ref[...])
    v_ref[...] = jnp.dot(d8 - p8, G3_ref[...],
                         preferred_element_type=jnp.float32)


def _tc1(h_p, pos8, A1, b1, A2, b2, G3, AH, bfr):
    full = lambda r, c: pl.BlockSpec((r, c), lambda i: (0, 0))
    return pl.pallas_call(
        _tc1_body,
        grid=(NBLK,),
        in_specs=[pl.BlockSpec((ROWB, C), lambda i: (i, 0)),
                  pl.BlockSpec((ROWB, 8), lambda i: (i, 0)),
                  full(C, C), full(1, C), full(C, 8), full(1, 8),
                  full(8, C), full(C, C), full(1, C)],
        out_specs=[pl.BlockSpec((ROWB, C), lambda i: (i, 0)),
                   pl.BlockSpec((ROWB, C), lambda i: (i, 0))],
        out_shape=[jax.ShapeDtypeStruct((NPAD, C), jnp.float32),
                   jax.ShapeDtypeStruct((NPAD, C), jnp.float32)],
    )(h_p, pos8, A1, b1, A2, b2, G3, AH, bfr)


_PERM_DN = lax.GatherDimensionNumbers(offset_dims=(), collapsed_slice_dims=(0,),
                                      start_index_map=(0,))


def _perm(x, idx):
    return lax.gather(x, idx[:, None], _PERM_DN, slice_sizes=(1,),
                      mode=lax.GatherScatterMode.PROMISE_IN_BOUNDS)


def _or_all(ms):
    while len(ms) > 1:
        ms = [a | b for a, b in zip(ms[::2], ms[1::2])] + (
            [ms[-1]] if len(ms) % 2 else [])
    return ms[0]


def _sc_body(u_t, edge_hbm, out_hbm, uflat, acc_a, acc_b, eb, sems):
    wid = lax.axis_index("s") * 2 + lax.axis_index("c")

    pltpu.sync_copy(u_t.at[wid], uflat)

    neg = jnp.full((16,), -jnp.inf, jnp.float32)

    def init_acc(i, _):
        acc_a[pl.ds(i * 16, 16)] = neg
        acc_b[pl.ds(i * 16, 16)] = neg
        return 0

    lax.fori_loop(0, NTW // 16, init_acc, 0)

    pltpu.make_async_copy(edge_hbm.at[:, pl.ds(0, CH)], eb.at[0],
                          sems.at[0]).start()
    pltpu.make_async_copy(edge_hbm.at[:, pl.ds(CH, CH)], eb.at[1],
                          sems.at[1]).start()

    def chunk_body(ci, _):
        p = ci % 2
        pltpu.make_async_copy(edge_hbm.at[:, pl.ds(ci * CH, CH)], eb.at[p],
                              sems.at[p]).wait()

        def blk_body(q, _):
            info = []
            for gg in range(GPB):
                g = q * GPB + gg
                acc = acc_a if gg % 2 == 0 else acc_b
                s16 = eb[p, 0, pl.ds(g * 16, 16)]
                d16 = eb[p, 1, pl.ds(g * 16, 16)]
                so = s16 * CPW
                do = d16 * CPW
                uvs = [plsc.load_gather(uflat, [so + c])
                       for c in range(CPW)]
                idxs = [do + c for c in range(CPW)]
                for c in range(CPW):
                    av = plsc.load_gather(acc, [idxs[c]])
                    plsc.store_scatter(acc, [idxs[c]], uvs[c],
                                       mask=uvs[c] > av)
                info.append((acc, idxs, uvs))

            stills = []
            for acc, idxs, uvs in info:
                for c in range(CPW):
                    av2 = plsc.load_gather(acc, [idxs[c]])
                    stills.append(uvs[c] > av2)
            cnt0 = plsc.all_reduce_population_count(_or_all(stills))[0]

            @pl.when(cnt0 > 0)
            def _():
                def repair(_cnt):
                    st2 = []
                    for acc, idxs, uvs in info:
                        for c in range(CPW):
                            av = plsc.load_gather(acc, [idxs[c]])
                            plsc.store_scatter(acc, [idxs[c]], uvs[c],
                                               mask=uvs[c] > av)
                            av2 = plsc.load_gather(acc, [idxs[c]])
                            st2.append(uvs[c] > av2)
                    return plsc.all_reduce_population_count(_or_all(st2))[0]

                lax.while_loop(lambda cnt: cnt > 0, repair, jnp.int32(1))

            return 0

        lax.fori_loop(0, NBL, blk_body, 0)

        @pl.when(ci + 2 < NCH)
        def _():
            pltpu.make_async_copy(edge_hbm.at[:, pl.ds((ci + 2) * CH, CH)],
                                  eb.at[p], sems.at[p]).start()

        return 0

    lax.fori_loop(0, NCH, chunk_body, 0)

    def merge(i, _):
        sl = pl.ds(i * 16, 16)
        acc_a[sl] = jnp.maximum(acc_a[sl], acc_b[sl])
        return 0

    lax.fori_loop(0, NTW // 16, merge, 0)
    pltpu.sync_copy(acc_a, out_hbm.at[wid])


def _sc_segmax(u_t, edge_index):
    mesh = plsc.VectorSubcoreMesh(core_axis_name="c", subcore_axis_name="s")
    f = pl.kernel(
        _sc_body,
        out_type=jax.ShapeDtypeStruct((NW, NTW), jnp.float32),
        mesh=mesh,
        scratch_types=[
            pltpu.VMEM((NTW,), jnp.float32),
            pltpu.VMEM((NTW,), jnp.float32),
            pltpu.VMEM((NTW,), jnp.float32),
            pltpu.VMEM((2, 2, CH), jnp.int32),
            pltpu.SemaphoreType.DMA((2,)),
        ],
        compiler_params=pltpu.CompilerParams(needs_layout_passes=False),
    )
    return f(u_t, edge_index)


def _tc2a_body(smax_ref, v_ref, h_ref, noise_ref, G1_ref, c1_ref, G2_ref,
               c2_ref, ns_ref, hh_ref, sums_ref):
    i = pl.program_id(0)
    sm = smax_ref[...]
    agg = jnp.where(jnp.isneginf(sm), 0.0, _leaky(sm + v_ref[...]))
    t = _leaky(jnp.dot(agg, G1_ref[...], preferred_element_type=jnp.float32)
               + c1_ref[...])
    out = (jnp.dot(t, G2_ref[...], preferred_element_type=jnp.float32)
           + c2_ref[...])
    hh = _leaky(h_ref[...] + out + noise_ref[...] * ns_ref[0, 0])
    rows = i * ROWB + lax.broadcasted_iota(jnp.int32, (ROWB, 1), 0)
    hh = jnp.where(rows < N, hh, 0.0)
    hh_ref[...] = hh

    s1 = jnp.sum(hh, axis=0, keepdims=True)
    s2 = jnp.sum(hh * hh, axis=0, keepdims=True)

    @pl.when(i == 0)
    def _():
        sums_ref[...] = jnp.zeros_like(sums_ref)

    sums_ref[0:1, :] += s1
    sums_ref[1:2, :] += s2


def _tc2a(smax, v, h_p, noise_p, G1, c1, G2, c2, ns):
    full = lambda r, c: pl.BlockSpec((r, c), lambda i: (0, 0))
    rb = pl.BlockSpec((ROWB, C), lambda i: (i, 0))
    return pl.pallas_call(
        _tc2a_body,
        grid=(NBLK,),
        in_specs=[rb, rb, rb, rb, full(C, C), full(1, C), full(C, C),
                  full(1, C), full(1, 1)],
        out_specs=[rb, full(8, C)],
        out_shape=[jax.ShapeDtypeStruct((NPAD, C), jnp.float32),
                   jax.ShapeDtypeStruct((8, C), jnp.float32)],
    )(smax, v, h_p, noise_p, G1, c1, G2, c2, ns)


def _tc2b_body(hh_ref, sums_ref, style_ref, Wa_ref, ba_ref, o_ref):
    s1 = sums_ref[0:1, :]
    s2 = sums_ref[1:2, :]
    mean = s1 * (1.0 / N)
    var = s2 * (1.0 / N) - mean * mean
    inv = lax.rsqrt(var + 1e-5)
    st = (jnp.dot(style_ref[...], Wa_ref[...],
                  preferred_element_type=jnp.float32) + ba_ref[...])
    gamma = st[:, :C]
    beta = st[:, C:]
    o_ref[...] = gamma * ((hh_ref[...] - mean) * inv) + beta


def _tc2b(hh, sums, style_p, WaT, ba):
    full = lambda r, c: pl.BlockSpec((r, c), lambda i: (0, 0))
    rb = pl.BlockSpec((ROWB, C), lambda i: (i, 0))
    return pl.pallas_call(
        _tc2b_body,
        grid=(NBLK,),
        in_specs=[rb, full(8, C), rb, full(C, 2 * C), full(1, 2 * C)],
        out_specs=rb,
        out_shape=jax.ShapeDtypeStruct((NPAD, C), jnp.float32),
    )(hh, sums, style_p, WaT, ba)


@jax.jit
def kernel(h, pos, style, noise, W1h, b1h, W2h, b2h, Wf, bf, W1g, b1g, W2g,
           b2g, W_aff, b_aff, noise_strength, edge_index):
    pad = NPAD - N
    h_p = jnp.pad(h, ((0, pad), (0, 0)))
    pos8 = jnp.pad(pos, ((0, pad), (0, 5)))
    noise_p = jnp.pad(noise, ((0, pad), (0, 0)))
    style_p = jnp.pad(style, ((0, pad), (0, 0)))

    A1 = W1h.T
    b1 = b1h.reshape(1, C)
    A2 = jnp.pad(W2h.T, ((0, 0), (0, 5)))
    b2 = jnp.pad(b2h, (0, 5)).reshape(1, 8)
    G3 = jnp.pad(Wf[:, :3].T, ((0, 5), (0, 0)))
    AH = Wf[:, 3:].T
    bfr = bf.reshape(1, C)
    G1 = W1g.T
    c1 = b1g.reshape(1, C)
    G2 = W2g.T
    c2 = b2g.reshape(1, C)
    WaT = W_aff.T
    ba = b_aff.reshape(1, 2 * C)
    ns = noise_strength.reshape(1, 1)

    u, v = _tc1(h_p, pos8, A1, b1, A2, b2, G3, AH, bfr)

    u_t = (jnp.pad(u, ((0, NT - NPAD), (0, 0)))
           .reshape(NT, NW, CPW).transpose(1, 0, 2).reshape(NW, NTW))
    smax_t = _sc_segmax(u_t, edge_index)
    smax = (smax_t.reshape(NW, NT, CPW).transpose(1, 0, 2)
            .reshape(NT, C)[:NPAD])

    hh, sums = _tc2a(smax, v, h_p, noise_p, G1, c1, G2, c2, ns)
    final = _tc2b(hh, sums, style_p, WaT, ba)
    return final[:N]
